# baseline (speedup 1.0000x reference)
import functools

import jax
import jax.numpy as jnp
from jax import lax
from jax.experimental import pallas as pl
from jax.experimental.pallas import tpu as pltpu
from jax.experimental.pallas import tpu_sc as plsc

_B = 2048
_BLK = 256
_K = 512
_D = 256

_NC, _NS = 2, 16
_NW = _NC * _NS
_BPW = (2 * _B) // _NW


def _top2_block(x_ref, emb_ref, i1_ref, i2_ref):
    x = x_ref[...]
    emb = emb_ref[...]

    esq = jnp.sum(emb * emb, axis=0)
    dots = jax.lax.dot_general(
        x, emb, (((1,), (0,)), ((), ())),
        precision=jax.lax.Precision.HIGHEST,
        preferred_element_type=jnp.float32)
    s = esq[None, :] - 2.0 * dots

    kidx = jax.lax.broadcasted_iota(jnp.int32, s.shape, 1)
    m1 = jnp.min(s, axis=1, keepdims=True)
    i1 = jnp.min(jnp.where(s == m1, kidx, _K), axis=1)
    s2 = jnp.where(kidx == i1[:, None], jnp.inf, s)
    m2 = jnp.min(s2, axis=1, keepdims=True)
    i2 = jnp.min(jnp.where(s2 == m2, kidx, _K), axis=1)
    i1_ref[0, 0, :] = i1
    i2_ref[0, 0, :] = i2


def _tc_top2(x, weight):
    nblk = _B // _BLK
    i1, i2 = pl.pallas_call(
        _top2_block,
        grid=(nblk,),
        in_specs=[
            pl.BlockSpec((_BLK, _D), lambda i: (i, 0)),
            pl.BlockSpec((_D, _K), lambda i: (0, 0)),
        ],
        out_specs=[
            pl.BlockSpec((1, 1, _BLK), lambda i: (i, 0, 0)),
            pl.BlockSpec((1, 1, _BLK), lambda i: (i, 0, 0)),
        ],
        out_shape=[
            jax.ShapeDtypeStruct((nblk, 1, _BLK), jnp.int32),
            jax.ShapeDtypeStruct((nblk, 1, _BLK), jnp.int32),
        ],
    )(x, weight)
    return i1.reshape(_B), i2.reshape(_B)


def _sc_gather_body(table_hbm, idx_hbm, out_hbm, idx_v, rows_v, sem):
    wid = lax.axis_index("s") * _NC + lax.axis_index("c")
    base = wid * _BPW
    pltpu.sync_copy(idx_hbm.at[pl.ds(base, _BPW)], idx_v)
    pltpu.async_copy(table_hbm.at[idx_v], rows_v, sem).wait()
    pltpu.sync_copy(rows_v, out_hbm.at[pl.ds(base, _BPW)])


def _sc_gather(table, idx):
    k = functools.partial(
        pl.kernel,
        mesh=plsc.VectorSubcoreMesh(core_axis_name="c", subcore_axis_name="s"),
        out_type=jax.ShapeDtypeStruct((2 * _B, _D), jnp.float32),
        scratch_types=[
            pltpu.VMEM((_BPW,), jnp.int32),
            pltpu.VMEM((_BPW, _D), jnp.float32),
            pltpu.SemaphoreType.DMA,
        ],
    )(_sc_gather_body)
    return k(table, idx)


def _refine_block(x_ref, e1_ref, e2_ref, i1_ref, i2_ref, out_ref):
    x = x_ref[...]
    e1 = e1_ref[...]
    e2 = e2_ref[...]
    i1 = i1_ref[0, 0, :]
    i2 = i2_ref[0, 0, :]
    d1 = jnp.sum((x - e1) ** 2, axis=1)
    d2 = jnp.sum((x - e2) ** 2, axis=1)
    pick1 = (d1 < d2) | ((d1 == d2) & (i1 < i2))
    out_ref[...] = jnp.where(pick1[:, None], e1, e2)


def _tc_refine(x, e1, e2, i1, i2):
    nblk = _B // _BLK
    return pl.pallas_call(
        _refine_block,
        grid=(nblk,),
        in_specs=[
            pl.BlockSpec((_BLK, _D), lambda i: (i, 0)),
            pl.BlockSpec((_BLK, _D), lambda i: (i, 0)),
            pl.BlockSpec((_BLK, _D), lambda i: (i, 0)),
            pl.BlockSpec((1, 1, _BLK), lambda i: (i, 0, 0)),
            pl.BlockSpec((1, 1, _BLK), lambda i: (i, 0, 0)),
        ],
        out_specs=pl.BlockSpec((_BLK, _D), lambda i: (i, 0)),
        out_shape=jax.ShapeDtypeStruct((_B, _D), jnp.float32),
    )(x, e1, e2, i1.reshape(_B // _BLK, 1, _BLK), i2.reshape(_B // _BLK, 1, _BLK))


@jax.jit
def kernel(x, weight):
    i1, i2 = _tc_top2(x, weight)
    idx = jnp.concatenate([i1, i2])
    rows = _sc_gather(weight.T, idx)
    e1, e2 = rows[:_B], rows[_B:]
    return _tc_refine(x, e1, e2, i1, i2)


# speedup vs baseline: 1.2112x; 1.2112x over previous
import functools

import jax
import jax.numpy as jnp
from jax import lax
from jax.experimental import pallas as pl
from jax.experimental.pallas import tpu as pltpu
from jax.experimental.pallas import tpu_sc as plsc

_B = 2048
_BLK = 256
_K = 512
_D = 256

_NC, _NS = 2, 16
_NW = _NC * _NS
_BPW = _B // _NW


def _pick_block(x_ref, emb_ref, embT_ref, idx_ref):
    x = x_ref[...]
    emb = emb_ref[...]
    embT = embT_ref[...]

    esq = jnp.sum(emb * emb, axis=0)
    dots = jax.lax.dot_general(
        x, emb, (((1,), (0,)), ((), ())),
        precision=jax.lax.Precision.HIGHEST,
        preferred_element_type=jnp.float32)
    s = esq[None, :] - 2.0 * dots

    kidx = jax.lax.broadcasted_iota(jnp.int32, s.shape, 1)
    m1 = jnp.min(s, axis=1, keepdims=True)
    i1 = jnp.min(jnp.where(s == m1, kidx, _K), axis=1)
    s2 = jnp.where(kidx == i1[:, None], jnp.inf, s)
    m2 = jnp.min(s2, axis=1, keepdims=True)
    i2 = jnp.min(jnp.where(s2 == m2, kidx, _K), axis=1)

    oh1 = (kidx == i1[:, None]).astype(jnp.float32)
    oh2 = (kidx == i2[:, None]).astype(jnp.float32)
    e1 = jax.lax.dot_general(
        oh1, embT, (((1,), (0,)), ((), ())),
        precision=jax.lax.Precision.HIGHEST,
        preferred_element_type=jnp.float32)
    e2 = jax.lax.dot_general(
        oh2, embT, (((1,), (0,)), ((), ())),
        precision=jax.lax.Precision.HIGHEST,
        preferred_element_type=jnp.float32)

    d1 = jnp.sum((x - e1) ** 2, axis=1)
    d2 = jnp.sum((x - e2) ** 2, axis=1)
    pick1 = (d1 < d2) | ((d1 == d2) & (i1 < i2))
    idx_ref[0, 0, :] = jnp.where(pick1, i1, i2)


def _tc_pick(x, weight, weight_t):
    nblk = _B // _BLK
    idx = pl.pallas_call(
        _pick_block,
        grid=(nblk,),
        in_specs=[
            pl.BlockSpec((_BLK, _D), lambda i: (i, 0)),
            pl.BlockSpec((_D, _K), lambda i: (0, 0)),
            pl.BlockSpec((_K, _D), lambda i: (0, 0)),
        ],
        out_specs=pl.BlockSpec((1, 1, _BLK), lambda i: (i, 0, 0)),
        out_shape=jax.ShapeDtypeStruct((nblk, 1, _BLK), jnp.int32),
    )(x, weight, weight_t)
    return idx.reshape(_B)


def _sc_gather_body(table_hbm, idx_hbm, out_hbm, idx_v, rows_v, sem):
    wid = lax.axis_index("s") * _NC + lax.axis_index("c")
    base = wid * _BPW
    pltpu.sync_copy(idx_hbm.at[pl.ds(base, _BPW)], idx_v)
    pltpu.async_copy(table_hbm.at[idx_v], rows_v, sem).wait()
    pltpu.sync_copy(rows_v, out_hbm.at[pl.ds(base, _BPW)])


def _sc_gather(table, idx):
    k = functools.partial(
        pl.kernel,
        mesh=plsc.VectorSubcoreMesh(core_axis_name="c", subcore_axis_name="s"),
        out_type=jax.ShapeDtypeStruct((_B, _D), jnp.float32),
        scratch_types=[
            pltpu.VMEM((_BPW,), jnp.int32),
            pltpu.VMEM((_BPW, _D), jnp.float32),
            pltpu.SemaphoreType.DMA,
        ],
    )(_sc_gather_body)
    return k(table, idx)


@jax.jit
def kernel(x, weight):
    weight_t = weight.T
    idx = _tc_pick(x, weight, weight_t)
    return _sc_gather(weight_t, idx)


# speedup vs baseline: 1.2121x; 1.0008x over previous
import functools

import jax
import jax.numpy as jnp
from jax import lax
from jax.experimental import pallas as pl
from jax.experimental.pallas import tpu as pltpu
from jax.experimental.pallas import tpu_sc as plsc

_B = 2048
_HALF = _B // 2
_BLK = 256
_K = 512
_D = 256

_NC, _NS = 2, 16
_NW = _NC * _NS
_BPW = _HALF // _NW


def _scores_top2(x, emb):
    esq = jnp.sum(emb * emb, axis=0)
    dots = jax.lax.dot_general(
        x, emb, (((1,), (0,)), ((), ())),
        precision=jax.lax.Precision.HIGHEST,
        preferred_element_type=jnp.float32)
    s = esq[None, :] - 2.0 * dots

    kidx = jax.lax.broadcasted_iota(jnp.int32, s.shape, 1)
    m1 = jnp.min(s, axis=1, keepdims=True)
    i1 = jnp.min(jnp.where(s == m1, kidx, _K), axis=1)
    s2 = jnp.where(kidx == i1[:, None], jnp.inf, s)
    m2 = jnp.min(s2, axis=1, keepdims=True)
    i2 = jnp.min(jnp.where(s2 == m2, kidx, _K), axis=1)
    return kidx, i1, i2


def _onehot_rows(kidx, i, embT):
    oh = (kidx == i[:, None]).astype(jnp.float32)
    return jax.lax.dot_general(
        oh, embT, (((1,), (0,)), ((), ())),
        precision=jax.lax.Precision.HIGHEST,
        preferred_element_type=jnp.float32)


def _pick_block(x_ref, emb_ref, embT_ref, idx_ref):
    x = x_ref[...]
    kidx, i1, i2 = _scores_top2(x, emb_ref[...])
    e1 = _onehot_rows(kidx, i1, embT_ref[...])
    e2 = _onehot_rows(kidx, i2, embT_ref[...])
    d1 = jnp.sum((x - e1) ** 2, axis=1)
    d2 = jnp.sum((x - e2) ** 2, axis=1)
    pick1 = (d1 < d2) | ((d1 == d2) & (i1 < i2))
    idx_ref[0, 0, :] = jnp.where(pick1, i1, i2)


def _full_block(x_ref, emb_ref, embT_ref, out_ref):
    x = x_ref[...]
    kidx, i1, i2 = _scores_top2(x, emb_ref[...])
    e1 = _onehot_rows(kidx, i1, embT_ref[...])
    e2 = _onehot_rows(kidx, i2, embT_ref[...])
    d1 = jnp.sum((x - e1) ** 2, axis=1)
    d2 = jnp.sum((x - e2) ** 2, axis=1)
    pick1 = (d1 < d2) | ((d1 == d2) & (i1 < i2))
    out_ref[...] = jnp.where(pick1[:, None], e1, e2)


def _tc_pick(x, weight, weight_t):
    nblk = x.shape[0] // _BLK
    idx = pl.pallas_call(
        _pick_block,
        grid=(nblk,),
        in_specs=[
            pl.BlockSpec((_BLK, _D), lambda i: (i, 0)),
            pl.BlockSpec((_D, _K), lambda i: (0, 0)),
            pl.BlockSpec((_K, _D), lambda i: (0, 0)),
        ],
        out_specs=pl.BlockSpec((1, 1, _BLK), lambda i: (i, 0, 0)),
        out_shape=jax.ShapeDtypeStruct((nblk, 1, _BLK), jnp.int32),
    )(x, weight, weight_t)
    return idx.reshape(x.shape[0])


def _tc_full(x, weight, weight_t):
    nblk = x.shape[0] // _BLK
    return pl.pallas_call(
        _full_block,
        grid=(nblk,),
        in_specs=[
            pl.BlockSpec((_BLK, _D), lambda i: (i, 0)),
            pl.BlockSpec((_D, _K), lambda i: (0, 0)),
            pl.BlockSpec((_K, _D), lambda i: (0, 0)),
        ],
        out_specs=pl.BlockSpec((_BLK, _D), lambda i: (i, 0)),
        out_shape=jax.ShapeDtypeStruct((x.shape[0], _D), jnp.float32),
    )(x, weight, weight_t)


def _sc_gather_body(table_hbm, idx_hbm, out_hbm, idx_v, rows_v, sem):
    wid = lax.axis_index("s") * _NC + lax.axis_index("c")
    base = wid * _BPW
    pltpu.sync_copy(idx_hbm.at[pl.ds(base, _BPW)], idx_v)
    pltpu.async_copy(table_hbm.at[idx_v], rows_v, sem).wait()
    pltpu.sync_copy(rows_v, out_hbm.at[pl.ds(base, _BPW)])


def _sc_gather(table, idx):
    k = functools.partial(
        pl.kernel,
        mesh=plsc.VectorSubcoreMesh(core_axis_name="c", subcore_axis_name="s"),
        out_type=jax.ShapeDtypeStruct((_HALF, _D), jnp.float32),
        scratch_types=[
            pltpu.VMEM((_BPW,), jnp.int32),
            pltpu.VMEM((_BPW, _D), jnp.float32),
            pltpu.SemaphoreType.DMA,
        ],
    )(_sc_gather_body)
    return k(table, idx)


@jax.jit
def kernel(x, weight):
    weight_t = weight.T
    xb = x[_HALF:]
    idx_b = _tc_pick(xb, weight, weight_t)
    out_b = _sc_gather(weight_t, idx_b)
    out_a = _tc_full(x[:_HALF], weight, weight_t)
    return jnp.concatenate([out_a, out_b], axis=0)
